# Initial kernel scaffold; baseline (speedup 1.0000x reference)
import jax
import jax.numpy as jnp
from jax.experimental import pallas as pl


def kernel(x, cluster_centers):
    raise NotImplementedError("write your pallas kernel here")


# speedup vs baseline: 2.1767x; 2.1767x over previous
import functools

import jax
import jax.numpy as jnp
from jax import lax
from jax.experimental import pallas as pl
from jax.experimental.pallas import tpu as pltpu
from jax.experimental.pallas import tpu_sc as plsc

N_CLUSTERS = 512
D = 32
N_TOKENS = 16 * 4096


_BS = 2048
_GRID = N_TOKENS // _BS


def _argmin_body(x_ref, c_ref, idx_ref):
    x = x_ref[...]
    c = c_ref[...]
    xc = lax.dot_general(
        x, c, (((1,), (1,)), ((), ())),
        preferred_element_type=jnp.float32,
        precision=lax.Precision.DEFAULT)
    c2 = jnp.sum(c * c, axis=1)
    scores = 2.0 * xc - c2[None, :]
    idx_ref[0, 0, :] = jnp.argmax(scores, axis=1).astype(jnp.int32)


def _nearest_idx(xr, centers):
    return pl.pallas_call(
        _argmin_body,
        grid=(_GRID,),
        in_specs=[
            pl.BlockSpec((_BS, D), lambda i: (i, 0)),
            pl.BlockSpec((N_CLUSTERS, D), lambda i: (0, 0)),
        ],
        out_specs=pl.BlockSpec((1, 1, _BS), lambda i: (i, 0, 0)),
        out_shape=jax.ShapeDtypeStruct((_GRID, 1, _BS), jnp.int32),
    )(xr, centers)


_NC, _NS = 2, 16
_NW = _NC * _NS
_BPW = N_TOKENS // _NW
_CH = 128
_NCH = _BPW // _CH

@functools.lru_cache(maxsize=None)
def _gather_fn():
    mesh = plsc.VectorSubcoreMesh(
        core_axis_name="c", subcore_axis_name="s",
        num_cores=_NC, num_subcores=_NS)

    @functools.partial(
        pl.kernel,
        mesh=mesh,
        out_type=jax.ShapeDtypeStruct((N_TOKENS, D), jnp.float32),
        scratch_types=[
            pltpu.VMEM((_NCH, _CH), jnp.int32),
            pltpu.VMEM((_BPW, D), jnp.float32),
            pltpu.SemaphoreType.DMA,
        ],
        compiler_params=pltpu.CompilerParams(use_tc_tiling_on_sc=False),
    )
    def _gather_rows(table_hbm, idx_hbm, out_hbm, idx_v, rows_v, sem):
        wid = lax.axis_index("s") * _NC + lax.axis_index("c")
        base = wid * _BPW
        pltpu.sync_copy(idx_hbm.at[pl.ds(wid * _NCH, _NCH)], idx_v)
        copies = []
        for j in range(_NCH):
            copies.append(
                pltpu.async_copy(
                    table_hbm.at[idx_v.at[j]],
                    rows_v.at[pl.ds(j * _CH, _CH)],
                    sem))
        for cp in copies:
            cp.wait()
        pltpu.sync_copy(rows_v, out_hbm.at[pl.ds(base, _BPW)])

    return _gather_rows


def kernel(x, cluster_centers):
    xr = x.reshape(N_TOKENS, D)
    idx = _nearest_idx(xr, cluster_centers)
    idx2 = idx.reshape(_NW * _NCH, _CH)
    out = _gather_fn()(cluster_centers, idx2)
    return out.reshape(x.shape)
